# Initial kernel scaffold; baseline (speedup 1.0000x reference)
import jax
import jax.numpy as jnp
from jax.experimental import pallas as pl


def kernel(labels, mean, log_var):
    raise NotImplementedError("write your pallas kernel here")


# speedup vs baseline: 102.4389x; 102.4389x over previous
import dataclasses

import jax
import jax.numpy as jnp
from jax import lax
from jax.experimental import pallas as pl
from jax.experimental.pallas import tpu as pltpu
from jax.experimental.pallas import tpu_sc as plsc

B = 16384
F = 26
K = 1000
N = B * F
NW = 32
CHUNK = N // NW
STEPS = CHUNK // 16
TBL = F * K


def _gather_kernel(labels_hbm, mean_hbm, logvar_hbm, out_m_hbm, out_lv_hbm,
                   lab_v, mean_v, logvar_v, om_v, olv_v):
    wid = lax.axis_index("subcore") * 2 + lax.axis_index("core")
    base = wid * CHUNK

    pltpu.sync_copy(labels_hbm.at[pl.ds(base, CHUNK)], lab_v)
    pltpu.sync_copy(mean_hbm, mean_v)
    pltpu.sync_copy(logvar_hbm, logvar_v)

    off0 = lax.iota(jnp.int32, 16) * K

    def body(t, off):
        p = pl.multiple_of(t * 16, 16)
        idx = lab_v[pl.ds(p, 16)] + off
        om_v[pl.ds(p, 16)] = plsc.load_gather(mean_v, [idx])
        olv_v[pl.ds(p, 16)] = plsc.load_gather(logvar_v, [idx])
        nxt = off + 16 * K
        return jnp.where(nxt >= F * K, nxt - F * K, nxt)

    lax.fori_loop(0, STEPS, body, off0, unroll=4)

    pltpu.sync_copy(om_v, out_m_hbm.at[pl.ds(base, CHUNK)])
    pltpu.sync_copy(olv_v, out_lv_hbm.at[pl.ds(base, CHUNK)])


@jax.jit
def kernel(labels, mean, log_var):
    mesh = plsc.VectorSubcoreMesh(core_axis_name="core",
                                  subcore_axis_name="subcore")
    cp = pltpu.CompilerParams()
    if "needs_layout_passes" in pltpu.CompilerParams.__dataclass_fields__:
        cp = dataclasses.replace(cp, needs_layout_passes=False)
    run = pl.kernel(
        _gather_kernel,
        out_type=(jax.ShapeDtypeStruct((N,), jnp.float32),
                  jax.ShapeDtypeStruct((N,), jnp.float32)),
        mesh=mesh,
        scratch_types=[
            pltpu.VMEM((CHUNK,), jnp.int32),
            pltpu.VMEM((TBL,), jnp.float32),
            pltpu.VMEM((TBL,), jnp.float32),
            pltpu.VMEM((CHUNK,), jnp.float32),
            pltpu.VMEM((CHUNK,), jnp.float32),
        ],
        compiler_params=cp,
    )
    om, olv = run(labels.astype(jnp.int32).reshape(N),
                  mean.reshape(TBL), log_var.reshape(TBL))
    return om.reshape(B, F), olv.reshape(B, F)


# speedup vs baseline: 103.4676x; 1.0100x over previous
import dataclasses

import jax
import jax.numpy as jnp
from jax import lax
from jax.experimental import pallas as pl
from jax.experimental.pallas import tpu as pltpu
from jax.experimental.pallas import tpu_sc as plsc

B = 16384
F = 26
K = 1000
N = B * F
NW = 32
CHUNK = N // NW
STEPS = CHUNK // 16
TBL = F * K


PAT = 13
GROUPS = STEPS // PAT


def _gather_kernel(labels_hbm, mean_hbm, logvar_hbm, off_hbm,
                   out_m_hbm, out_lv_hbm,
                   lab_v, mean_v, logvar_v, off_v, om_v, olv_v,
                   sem_a, sem_b, sem_c, sem_d):
    wid = lax.axis_index("subcore") * 2 + lax.axis_index("core")
    base = wid * CHUNK

    ca = pltpu.async_copy(labels_hbm.at[pl.ds(base, CHUNK)], lab_v, sem_a)
    cb = pltpu.async_copy(mean_hbm, mean_v, sem_b)
    cc = pltpu.async_copy(logvar_hbm, logvar_v, sem_c)
    cd = pltpu.async_copy(off_hbm, off_v, sem_d)
    ca.wait(); cb.wait(); cc.wait(); cd.wait()

    @pl.loop(0, GROUPS)
    def _(g):
        gp = pl.multiple_of(g * PAT * 16, 16)
        for s in range(PAT):
            p = gp + s * 16
            idx = lab_v[pl.ds(p, 16)] + off_v[pl.ds(s * 16, 16)]
            om_v[pl.ds(p, 16)] = plsc.load_gather(mean_v, [idx])
            olv_v[pl.ds(p, 16)] = plsc.load_gather(logvar_v, [idx])

    pltpu.sync_copy(om_v, out_m_hbm.at[pl.ds(base, CHUNK)])
    pltpu.sync_copy(olv_v, out_lv_hbm.at[pl.ds(base, CHUNK)])


@jax.jit
def kernel(labels, mean, log_var):
    mesh = plsc.VectorSubcoreMesh(core_axis_name="core",
                                  subcore_axis_name="subcore")
    cp = pltpu.CompilerParams()
    if "needs_layout_passes" in pltpu.CompilerParams.__dataclass_fields__:
        cp = dataclasses.replace(cp, needs_layout_passes=False)
    run = pl.kernel(
        _gather_kernel,
        out_type=(jax.ShapeDtypeStruct((N,), jnp.float32),
                  jax.ShapeDtypeStruct((N,), jnp.float32)),
        mesh=mesh,
        scratch_types=[
            pltpu.VMEM((CHUNK,), jnp.int32),
            pltpu.VMEM((TBL,), jnp.float32),
            pltpu.VMEM((TBL,), jnp.float32),
            pltpu.VMEM((PAT * 16,), jnp.int32),
            pltpu.VMEM((CHUNK,), jnp.float32),
            pltpu.VMEM((CHUNK,), jnp.float32),
            pltpu.SemaphoreType.DMA,
            pltpu.SemaphoreType.DMA,
            pltpu.SemaphoreType.DMA,
            pltpu.SemaphoreType.DMA,
        ],
        compiler_params=cp,
    )
    off_pattern = (jnp.arange(PAT * 16, dtype=jnp.int32) % F) * K
    om, olv = run(labels.astype(jnp.int32).reshape(N),
                  mean.reshape(TBL), log_var.reshape(TBL), off_pattern)
    return om.reshape(B, F), olv.reshape(B, F)


# speedup vs baseline: 209.8613x; 2.0283x over previous
import dataclasses

import jax
import jax.numpy as jnp
from jax import lax
from jax.experimental import pallas as pl
from jax.experimental.pallas import tpu as pltpu
from jax.experimental.pallas import tpu_sc as plsc

B = 16384
F = 26
K = 1000
NW = 32
ROWS = B // NW
VB = 16


def _gather_kernel(lab_hbm, mean_hbm, logvar_hbm, out_m_hbm, out_lv_hbm,
                   lab_v, mean_v, logvar_v, om_v, olv_v,
                   sem_a, sem_b, sem_c):
    wid = lax.axis_index("subcore") * 2 + lax.axis_index("core")
    base = wid * ROWS

    ca = pltpu.async_copy(lab_hbm.at[:, pl.ds(base, ROWS)], lab_v, sem_a)
    cb = pltpu.async_copy(mean_hbm, mean_v, sem_b)
    cc = pltpu.async_copy(logvar_hbm, logvar_v, sem_c)
    ca.wait(); cb.wait(); cc.wait()

    @pl.loop(0, ROWS // (8 * VB))
    def _(c):
        i0 = pl.multiple_of(c * (8 * VB), 8 * VB)
        for j in range(F):
            row = jnp.full((VB,), j, jnp.int32)
            for v in range(8):
                sl = pl.ds(i0 + v * VB, VB)
                lab = lab_v[j, sl]
                om_v[j, sl] = plsc.load_gather(mean_v, [row, lab])
                olv_v[j, sl] = plsc.load_gather(logvar_v, [row, lab])

    da = pltpu.async_copy(om_v, out_m_hbm.at[:, pl.ds(base, ROWS)], sem_a)
    db = pltpu.async_copy(olv_v, out_lv_hbm.at[:, pl.ds(base, ROWS)], sem_b)
    da.wait(); db.wait()


@jax.jit
def kernel(labels, mean, log_var):
    mesh = plsc.VectorSubcoreMesh(core_axis_name="core",
                                  subcore_axis_name="subcore")
    cp = pltpu.CompilerParams(use_tc_tiling_on_sc=True)
    if "needs_layout_passes" in pltpu.CompilerParams.__dataclass_fields__:
        cp = dataclasses.replace(cp, needs_layout_passes=False)
    run = pl.kernel(
        _gather_kernel,
        out_type=(jax.ShapeDtypeStruct((F, B), jnp.float32),
                  jax.ShapeDtypeStruct((F, B), jnp.float32)),
        mesh=mesh,
        scratch_types=[
            pltpu.VMEM((F, ROWS), jnp.int32),
            pltpu.VMEM((F, K), jnp.float32),
            pltpu.VMEM((F, K), jnp.float32),
            pltpu.VMEM((F, ROWS), jnp.float32),
            pltpu.VMEM((F, ROWS), jnp.float32),
            pltpu.SemaphoreType.DMA,
            pltpu.SemaphoreType.DMA,
            pltpu.SemaphoreType.DMA,
        ],
        compiler_params=cp,
    )
    om_t, olv_t = run(labels.astype(jnp.int32).T, mean, log_var)
    return om_t.T, olv_t.T


# speedup vs baseline: 245.3492x; 1.1691x over previous
import dataclasses

import jax
import jax.numpy as jnp
from jax import lax
from jax.experimental import pallas as pl
from jax.experimental.pallas import tpu as pltpu
from jax.experimental.pallas import tpu_sc as plsc

B = 16384
F = 26
K = 1000
NW = 32
ROWS = B // NW
VB = 16


def _gather_kernel(lab_hbm, mean_hbm, logvar_hbm, out_m_hbm, out_lv_hbm,
                   lab_v, mean_v, logvar_v, om_v, olv_v,
                   sem_a, sem_b, sem_c):
    wid = lax.axis_index("subcore") * 2 + lax.axis_index("core")
    base = wid * ROWS

    ca = pltpu.async_copy(lab_hbm.at[:, pl.ds(base, ROWS)], lab_v, sem_a)
    cb = pltpu.async_copy(mean_hbm, mean_v, sem_b)
    cc = pltpu.async_copy(logvar_hbm, logvar_v, sem_c)
    ca.wait(); cb.wait(); cc.wait()

    def do_chunk(c):
        i0 = pl.multiple_of(c * (8 * VB), 8 * VB)
        for j in range(F):
            row = jnp.full((VB,), j, jnp.int32)
            sls = [pl.ds(i0 + v * VB, VB) for v in range(8)]
            labs = [lab_v[j, sl] for sl in sls]
            ms = [plsc.load_gather(mean_v, [row, lab]) for lab in labs]
            lvs = [plsc.load_gather(logvar_v, [row, lab]) for lab in labs]
            for sl, m in zip(sls, ms):
                om_v[j, sl] = m
            for sl, lv in zip(sls, lvs):
                olv_v[j, sl] = lv

    NCH = ROWS // (8 * VB)
    copies = []
    for c in range(NCH):
        do_chunk(c)
        cs = pl.ds(base + c * 8 * VB, 8 * VB)
        ls = pl.ds(c * 8 * VB, 8 * VB)
        copies.append(pltpu.async_copy(
            om_v.at[:, ls], out_m_hbm.at[:, cs], sem_a))
        copies.append(pltpu.async_copy(
            olv_v.at[:, ls], out_lv_hbm.at[:, cs], sem_b))
    for cp_ in copies:
        cp_.wait()


@jax.jit
def kernel(labels, mean, log_var):
    mesh = plsc.VectorSubcoreMesh(core_axis_name="core",
                                  subcore_axis_name="subcore")
    cp = pltpu.CompilerParams(use_tc_tiling_on_sc=True)
    if "needs_layout_passes" in pltpu.CompilerParams.__dataclass_fields__:
        cp = dataclasses.replace(cp, needs_layout_passes=False)
    run = pl.kernel(
        _gather_kernel,
        out_type=(jax.ShapeDtypeStruct((F, B), jnp.float32),
                  jax.ShapeDtypeStruct((F, B), jnp.float32)),
        mesh=mesh,
        scratch_types=[
            pltpu.VMEM((F, ROWS), jnp.int32),
            pltpu.VMEM((F, K), jnp.float32),
            pltpu.VMEM((F, K), jnp.float32),
            pltpu.VMEM((F, ROWS), jnp.float32),
            pltpu.VMEM((F, ROWS), jnp.float32),
            pltpu.SemaphoreType.DMA,
            pltpu.SemaphoreType.DMA,
            pltpu.SemaphoreType.DMA,
        ],
        compiler_params=cp,
    )
    om_t, olv_t = run(labels.astype(jnp.int32).T, mean, log_var)
    return om_t.T, olv_t.T


# speedup vs baseline: 286.2690x; 1.1668x over previous
import dataclasses

import jax
import jax.numpy as jnp
from jax import lax
from jax.experimental import pallas as pl
from jax.experimental.pallas import tpu as pltpu
from jax.experimental.pallas import tpu_sc as plsc

B = 16384
F = 26
K = 1000
NW = 32
ROWS = B // NW
VB = 16


def _gather_kernel(lab_hbm, mean_hbm, logvar_hbm, out_m_hbm, out_lv_hbm,
                   lab_v, mean_v, logvar_v, om_v, olv_v,
                   sem_a, sem_b, sem_c):
    wid = lax.axis_index("subcore") * 2 + lax.axis_index("core")
    base = wid * ROWS

    ca = pltpu.async_copy(lab_hbm.at[:, pl.ds(base, ROWS)], lab_v, sem_a)
    cb = pltpu.async_copy(mean_hbm, mean_v, sem_b)
    cc = pltpu.async_copy(logvar_hbm, logvar_v, sem_c)
    ca.wait(); cb.wait(); cc.wait()

    def do_chunk(c):
        i0 = pl.multiple_of(c * (8 * VB), 8 * VB)
        for j in range(F):
            row = jnp.full((VB,), j, jnp.int32)
            sls = [pl.ds(i0 + v * VB, VB) for v in range(8)]
            labs = [lab_v[j, sl] for sl in sls]
            ms = [plsc.load_gather(mean_v, [row, lab]) for lab in labs]
            lvs = [plsc.load_gather(logvar_v, [row, lab]) for lab in labs]
            for sl, m in zip(sls, ms):
                om_v[j, sl] = m
            for sl, lv in zip(sls, lvs):
                olv_v[j, sl] = lv

    NCH = ROWS // (8 * VB)

    @pl.loop(0, NCH)
    def _(c):
        do_chunk(c)
        cs = pl.ds(base + c * 8 * VB, 8 * VB)
        ls = pl.ds(c * 8 * VB, 8 * VB)
        pltpu.async_copy(om_v.at[:, ls], out_m_hbm.at[:, cs], sem_a)
        pltpu.async_copy(olv_v.at[:, ls], out_lv_hbm.at[:, cs], sem_b)

    pltpu.make_async_copy(om_v, out_m_hbm.at[:, pl.ds(base, ROWS)],
                          sem_a).wait()
    pltpu.make_async_copy(olv_v, out_lv_hbm.at[:, pl.ds(base, ROWS)],
                          sem_b).wait()


@jax.jit
def kernel(labels, mean, log_var):
    mesh = plsc.VectorSubcoreMesh(core_axis_name="core",
                                  subcore_axis_name="subcore")
    cp = pltpu.CompilerParams(use_tc_tiling_on_sc=True)
    if "needs_layout_passes" in pltpu.CompilerParams.__dataclass_fields__:
        cp = dataclasses.replace(cp, needs_layout_passes=False)
    run = pl.kernel(
        _gather_kernel,
        out_type=(jax.ShapeDtypeStruct((F, B), jnp.float32),
                  jax.ShapeDtypeStruct((F, B), jnp.float32)),
        mesh=mesh,
        scratch_types=[
            pltpu.VMEM((F, ROWS), jnp.int32),
            pltpu.VMEM((F, K), jnp.float32),
            pltpu.VMEM((F, K), jnp.float32),
            pltpu.VMEM((F, ROWS), jnp.float32),
            pltpu.VMEM((F, ROWS), jnp.float32),
            pltpu.SemaphoreType.DMA,
            pltpu.SemaphoreType.DMA,
            pltpu.SemaphoreType.DMA,
        ],
        compiler_params=cp,
    )
    om_t, olv_t = run(labels.astype(jnp.int32).T, mean, log_var)
    return om_t.T, olv_t.T


# speedup vs baseline: 384.3875x; 1.3427x over previous
import dataclasses

import jax
import jax.numpy as jnp
from jax import lax
from jax.experimental import pallas as pl
from jax.experimental.pallas import tpu as pltpu
from jax.experimental.pallas import tpu_sc as plsc

B = 16384
F = 26
K = 1000
VB = 16
GRP = 8
CHUNK = 16 * VB * GRP
NCH = B // CHUNK


def _gather_kernel(lab_hbm, mean_hbm, logvar_hbm, out_m_hbm, out_lv_hbm,
                   lab_v, mean_v, logvar_v, om_v, olv_v,
                   sem_a, sem_b, sem_c):
    wid = lax.axis_index("subcore") * 2 + lax.axis_index("core")

    @pl.when(wid < F)
    def _():
        col = pl.ds(wid, 1)
        ca = pltpu.async_copy(lab_hbm.at[col, :], lab_v, sem_a)
        cb = pltpu.async_copy(mean_hbm.at[col, :], mean_v, sem_b)
        cc = pltpu.async_copy(logvar_hbm.at[col, :], logvar_v, sem_c)
        cb.wait(); cc.wait(); ca.wait()

        row = jnp.zeros((VB,), jnp.int32)

        @pl.loop(0, NCH)
        def _(c):
            i0 = pl.multiple_of(c * CHUNK, CHUNK)
            for g in range(CHUNK // (GRP * VB)):
                b0 = i0 + g * GRP * VB
                sls = [pl.ds(b0 + v * VB, VB) for v in range(GRP)]
                labs = [lab_v[0, sl] for sl in sls]
                ms = [plsc.load_gather(mean_v, [row, lab]) for lab in labs]
                lvs = [plsc.load_gather(logvar_v, [row, lab]) for lab in labs]
                for sl, m in zip(sls, ms):
                    om_v[0, sl] = m
                for sl, lv in zip(sls, lvs):
                    olv_v[0, sl] = lv
            cs = pl.ds(c * CHUNK, CHUNK)
            pltpu.async_copy(om_v.at[:, cs], out_m_hbm.at[col, cs], sem_a)
            pltpu.async_copy(olv_v.at[:, cs], out_lv_hbm.at[col, cs], sem_b)

        pltpu.make_async_copy(om_v, out_m_hbm.at[col, :], sem_a).wait()
        pltpu.make_async_copy(olv_v, out_lv_hbm.at[col, :], sem_b).wait()


@jax.jit
def kernel(labels, mean, log_var):
    mesh = plsc.VectorSubcoreMesh(core_axis_name="core",
                                  subcore_axis_name="subcore")
    cp = pltpu.CompilerParams(use_tc_tiling_on_sc=True)
    if "needs_layout_passes" in pltpu.CompilerParams.__dataclass_fields__:
        cp = dataclasses.replace(cp, needs_layout_passes=False)
    run = pl.kernel(
        _gather_kernel,
        out_type=(jax.ShapeDtypeStruct((F, B), jnp.float32),
                  jax.ShapeDtypeStruct((F, B), jnp.float32)),
        mesh=mesh,
        scratch_types=[
            pltpu.VMEM((1, B), jnp.int32),
            pltpu.VMEM((1, K), jnp.float32),
            pltpu.VMEM((1, K), jnp.float32),
            pltpu.VMEM((1, B), jnp.float32),
            pltpu.VMEM((1, B), jnp.float32),
            pltpu.SemaphoreType.DMA,
            pltpu.SemaphoreType.DMA,
            pltpu.SemaphoreType.DMA,
        ],
        compiler_params=cp,
    )
    om_t, olv_t = run(labels.astype(jnp.int32).T, mean, log_var)
    return om_t.T, olv_t.T


# speedup vs baseline: 385.6656x; 1.0033x over previous
import dataclasses

import jax
import jax.numpy as jnp
from jax import lax
from jax.experimental import pallas as pl
from jax.experimental.pallas import tpu as pltpu
from jax.experimental.pallas import tpu_sc as plsc

B = 16384
F = 26
K = 1000
VB = 16
GRP = 8
CHUNK = 16 * VB * GRP
NCH = B // CHUNK


def _gather_kernel(lab_hbm, mean_hbm, logvar_hbm, out_m_hbm, out_lv_hbm,
                   lab_v, mean_v, logvar_v, om_v, olv_v,
                   sem_a, sem_b, sem_c):
    wid = lax.axis_index("subcore") * 2 + lax.axis_index("core")

    @pl.when(wid < F)
    def _():
        col = pl.ds(wid, 1)
        ca = pltpu.async_copy(lab_hbm.at[col, :], lab_v, sem_a)
        cb = pltpu.async_copy(mean_hbm.at[col, :], mean_v, sem_b)
        cc = pltpu.async_copy(logvar_hbm.at[col, :], logvar_v, sem_c)
        cb.wait(); cc.wait(); ca.wait()

        row = jnp.zeros((VB,), jnp.int32)

        @pl.loop(0, NCH)
        def _(c):
            i0 = pl.multiple_of(c * CHUNK, CHUNK)

            @pl.loop(0, CHUNK // (GRP * VB))
            def _(g):
                b0 = i0 + pl.multiple_of(g * GRP * VB, GRP * VB)
                sls = [pl.ds(b0 + v * VB, VB) for v in range(GRP)]
                labs = [lab_v[0, sl] for sl in sls]
                ms = [plsc.load_gather(mean_v, [row, lab]) for lab in labs]
                lvs = [plsc.load_gather(logvar_v, [row, lab]) for lab in labs]
                for sl, m in zip(sls, ms):
                    om_v[0, sl] = m
                for sl, lv in zip(sls, lvs):
                    olv_v[0, sl] = lv

            cs = pl.ds(c * CHUNK, CHUNK)
            pltpu.async_copy(om_v.at[:, cs], out_m_hbm.at[col, cs], sem_a)
            pltpu.async_copy(olv_v.at[:, cs], out_lv_hbm.at[col, cs], sem_b)

        pltpu.make_async_copy(om_v, out_m_hbm.at[col, :], sem_a).wait()
        pltpu.make_async_copy(olv_v, out_lv_hbm.at[col, :], sem_b).wait()


@jax.jit
def kernel(labels, mean, log_var):
    mesh = plsc.VectorSubcoreMesh(core_axis_name="core",
                                  subcore_axis_name="subcore")
    cp = pltpu.CompilerParams(use_tc_tiling_on_sc=True)
    if "needs_layout_passes" in pltpu.CompilerParams.__dataclass_fields__:
        cp = dataclasses.replace(cp, needs_layout_passes=False)
    run = pl.kernel(
        _gather_kernel,
        out_type=(jax.ShapeDtypeStruct((F, B), jnp.float32),
                  jax.ShapeDtypeStruct((F, B), jnp.float32)),
        mesh=mesh,
        scratch_types=[
            pltpu.VMEM((1, B), jnp.int32),
            pltpu.VMEM((1, K), jnp.float32),
            pltpu.VMEM((1, K), jnp.float32),
            pltpu.VMEM((1, B), jnp.float32),
            pltpu.VMEM((1, B), jnp.float32),
            pltpu.SemaphoreType.DMA,
            pltpu.SemaphoreType.DMA,
            pltpu.SemaphoreType.DMA,
        ],
        compiler_params=cp,
    )
    om_t, olv_t = run(labels.astype(jnp.int32).T, mean, log_var)
    return om_t.T, olv_t.T
